# Initial kernel scaffold; baseline (speedup 1.0000x reference)
import jax
import jax.numpy as jnp
from jax.experimental import pallas as pl


def kernel(vecs, codebook):
    raise NotImplementedError("write your pallas kernel here")


# speedup vs baseline: 1.2701x; 1.2701x over previous
import jax
import jax.numpy as jnp
from jax.experimental import pallas as pl

B_BLK = 512
M_GRP = 4


def _pq_kernel(vecs_ref, cb_ref, out_ref):
    K = cb_ref.shape[1]
    dsub = cb_ref.shape[2]
    for j in range(M_GRP):
        v = vecs_ref[:, j * dsub:(j + 1) * dsub]
        cb = cb_ref[j]
        cross = jnp.dot(v, cb.T, preferred_element_type=jnp.float32)
        v_sq = jnp.sum(v * v, axis=1, keepdims=True)
        c_sq = jnp.sum(cb * cb, axis=1)
        proba = -(v_sq - 2.0 * cross + c_sq[None, :])
        idx = jnp.argmax(proba, axis=1)
        k_iota = jax.lax.broadcasted_iota(jnp.int32, proba.shape, 1)
        onehot = (k_iota == idx[:, None]).astype(jnp.float32)
        out_ref[:, j * dsub:(j + 1) * dsub] = jnp.dot(
            onehot, cb, preferred_element_type=jnp.float32)


def kernel(vecs, codebook):
    B, D = vecs.shape
    M, K, dsub = codebook.shape
    grid = (M // M_GRP, B // B_BLK)
    return pl.pallas_call(
        _pq_kernel,
        grid=grid,
        in_specs=[
            pl.BlockSpec((B_BLK, M_GRP * dsub), lambda mg, i: (i, mg)),
            pl.BlockSpec((M_GRP, K, dsub), lambda mg, i: (mg, 0, 0)),
        ],
        out_specs=pl.BlockSpec((B_BLK, M_GRP * dsub), lambda mg, i: (i, mg)),
        out_shape=jax.ShapeDtypeStruct((B, D), jnp.float32),
    )(vecs, codebook)
